# baseline (speedup 1.0000x reference)
import jax
import jax.numpy as jnp
from jax import lax
from jax.experimental import pallas as pl
from jax.experimental.pallas import tpu as pltpu
from jax.experimental.pallas import tpu_sc as plsc

_B = 16384
_F = 26
_V = 100000
_D = 16
_DENSE = 13

_NC = 2
_NS = 16
_NW = _NC * _NS
_RPW = _B // _NW

_VFULL = _V // 128
_VTAIL = _V - _VFULL * 128
_VROWS = 12504
_VDBL = _VFULL // 2
_NBLK = _F * _VDBL


_NBUF = 4
_NIT = (_NBLK // _NW + _NBUF) // _NBUF


def _tr_body(embt_hbm, tail_hbm, out_hbm,
             vin0, vin1, vin2, vin3, vout0, vout1, vout2, vout3,
             rsem, wsem):
    wid = lax.axis_index("s") * _NC + lax.axis_index("c")
    lane = lax.iota(jnp.int32, 16)
    start = wid * _NBLK // _NW
    stop = (wid + 1) * _NBLK // _NW
    vin = [vin0, vin1, vin2, vin3]
    vout = [vout0, vout1, vout2, vout3]

    def issue_read(t, b):
        @pl.when(t < stop)
        def _():
            f = t // _VDBL
            c = lax.rem(t, _VDBL)
            pltpu.async_copy(
                embt_hbm.at[f, :, pl.ds(c * 256, 256)], vin[b], rsem)

    for b in range(_NBUF):
        issue_read(start + b, b)

    def ring(g, carry):
        t0 = start + g * _NBUF
        for b in range(_NBUF):
            t = t0 + b

            @pl.when(t < stop)
            def _(b=b, t=t):
                pltpu.make_async_copy(
                    embt_hbm.at[0, :, pl.ds(0, 256)], vin[b], rsem).wait()

                @pl.when(t - _NBUF >= start)
                def _():
                    pltpu.make_async_copy(
                        vout[b], out_hbm.at[0, pl.ds(0, 32), :], wsem).wait()

                for v in range(256):
                    cv = (lane + v) & 255
                    rv = lax.shift_right_logical(cv, 3)
                    ov = ((cv & 7) << 4) + lane
                    e = plsc.load_gather(vin[b], [lane, cv])
                    plsc.store_scatter(vout[b], [rv, ov], e)
                f = t // _VDBL
                c = lax.rem(t, _VDBL)
                pltpu.async_copy(
                    vout[b], out_hbm.at[f, pl.ds(c * 32, 32), :], wsem)
                issue_read(t + _NBUF, b)
        return carry

    lax.fori_loop(0, _NIT, ring, 0)
    for b in range(_NBUF):
        pltpu.make_async_copy(
            vout[b], out_hbm.at[0, pl.ds(0, 32), :], wsem).wait()

    @pl.when(wid < _F)
    def _():
        pltpu.async_copy(
            embt_hbm.at[wid, :, pl.ds(_VDBL * 256, 128)],
            vin0.at[:, pl.ds(0, 128)], rsem).wait()
        for v in range(128):
            cv = (lane + v) & 127
            rv = lax.shift_right_logical(cv, 3)
            ov = ((cv & 7) << 4) + lane
            e = plsc.load_gather(vin0, [lane, cv])
            plsc.store_scatter(vout0, [rv, ov], e)
        pltpu.async_copy(vout0.at[pl.ds(0, 16)],
                         out_hbm.at[wid, pl.ds(_VDBL * 32, 16), :],
                         wsem).wait()
        pltpu.async_copy(tail_hbm.at[wid], vin0.at[:, pl.ds(0, 128)],
                         rsem).wait()
        for v in range(128):
            cv = (lane + v) & 127
            rv = lax.shift_right_logical(cv, 3)
            ov = ((cv & 7) << 4) + lane
            e = plsc.load_gather(vin0, [lane, cv])
            plsc.store_scatter(vout0, [rv, ov], e)
        pltpu.async_copy(vout0.at[pl.ds(0, 8)],
                         out_hbm.at[wid, pl.ds(_VFULL * 16, 8), :],
                         wsem).wait()


_tr_call = pl.kernel(
    _tr_body,
    out_type=jax.ShapeDtypeStruct((_F, _VROWS, 128), jnp.float32),
    mesh=plsc.VectorSubcoreMesh(core_axis_name="c", subcore_axis_name="s"),
    compiler_params=pltpu.CompilerParams(
        needs_layout_passes=False, use_tc_tiling_on_sc=True),
    scratch_types=(
        [pltpu.VMEM((16, 256), jnp.float32) for _ in range(_NBUF)]
        + [pltpu.VMEM((32, 128), jnp.float32) for _ in range(_NBUF)]
        + [pltpu.SemaphoreType.DMA, pltpu.SemaphoreType.DMA]
    ),
)


_CB2 = 16
_NCHUNK2 = _RPW // _CB2
_G2 = _F * _CB2


def _so_body(emb_hbm, idx_hbm, out_hbm,
             idx_vm, ridx_vm, off_vm, emb_vm, out_vm, sem):
    wid = lax.axis_index("s") * _NC + lax.axis_index("c")
    lane = lax.iota(jnp.int32, 16)
    half = wid // 2
    sub = (wid % 2) * 4

    cps = []
    for f in range(_F):
        cps.append(pltpu.async_copy(
            idx_hbm.at[f, pl.ds(half * 8, 8), :], idx_vm.at[f], sem))
    for cp in cps:
        cp.wait()

    def chunk(c, carry):
        lr = sub + c // 8
        col = (c % 8) * 16
        for f in range(_F):
            v = idx_vm[f, lr, pl.ds(col, 16)]
            off_vm[f, :] = (v & 7) * 16
            ridx_vm[f, :] = lax.shift_right_logical(v, 3) + f * _VROWS
        copies = []
        for f in range(_F):
            copies.append(pltpu.async_copy(
                emb_hbm.at[ridx_vm.at[f]],
                emb_vm.at[pl.ds(f * _CB2, _CB2)], sem))
        for cp in copies:
            cp.wait()

        eres = jnp.zeros((16,), jnp.float32)
        for j in range(_CB2):
            acc_s = jnp.zeros((16,), jnp.float32)
            acc_q = jnp.zeros((16,), jnp.float32)
            jsplat = jnp.full((16,), j, jnp.int32)
            for f in range(_F):
                osp = plsc.load_gather(
                    off_vm, [jnp.full((16,), f, jnp.int32), jsplat])
                e = plsc.load_gather(
                    emb_vm, [jnp.full((16,), f * _CB2 + j, jnp.int32),
                             osp + lane])
                acc_s = acc_s + e
                acc_q = acc_q + e * e
            r = 0.5 * jnp.sum(acc_s * acc_s - acc_q)
            eres = jnp.where(lane == j, eres + r, eres)
        out_vm[c // 8, pl.ds((c % 8) * 16, 16)] = eres
        return carry

    lax.fori_loop(0, _NCHUNK2, chunk, 0)
    pltpu.sync_copy(out_vm, out_hbm.at[wid])


_so_call = pl.kernel(
    _so_body,
    out_type=jax.ShapeDtypeStruct((_NW, 8, 128), jnp.float32),
    mesh=plsc.VectorSubcoreMesh(core_axis_name="c", subcore_axis_name="s"),
    compiler_params=pltpu.CompilerParams(
        needs_layout_passes=False, use_tc_tiling_on_sc=True),
    scratch_types=[
        pltpu.VMEM((_F, 8, 128), jnp.int32),
        pltpu.VMEM((_F, _CB2), jnp.int32),
        pltpu.VMEM((_F, _CB2), jnp.int32),
        pltpu.VMEM((_G2, 128), jnp.float32),
        pltpu.VMEM((8, 128), jnp.float32),
        pltpu.SemaphoreType.DMA,
    ],
)


_CB1 = 64
_NCHUNK1 = _RPW // _CB1


def _fo_body(w1_hbm, idx_hbm, dx_hbm, wb_hbm, out_hbm,
             idx_vm, dx_vm, fidx_vm, w1_vm, wb_vm, out_vm, sem):
    wid = lax.axis_index("s") * _NC + lax.axis_index("c")
    lane = lax.iota(jnp.int32, 16)

    pltpu.sync_copy(wb_hbm, wb_vm)
    wv = wb_vm[...]
    wsplat = [jnp.sum(jnp.where(lane == k, wv, 0.0))
              for k in range(_DENSE + 1)]

    def chunk(c, carry):
        base = wid * _RPW + c * _CB1
        cps_in = []
        for f in range(_F):
            cps_in.append(pltpu.async_copy(
                idx_hbm.at[pl.ds(f * _B + base, _CB1)], idx_vm.at[f], sem))
        for k in range(_DENSE):
            cps_in.append(pltpu.async_copy(
                dx_hbm.at[pl.ds(k * _B + base, _CB1)], dx_vm.at[k], sem))
        for cp in cps_in:
            cp.wait()

        for f in range(_F):
            for g in range(_CB1 // 16):
                v = idx_vm[f, pl.ds(g * 16, 16)] + f * _V
                p = f * _CB1 + g * 16
                fidx_vm[p // 128, pl.ds(p % 128, 16)] = v

        copies = []
        for i in range(_F * _CB1 // 128):
            copies.append(pltpu.async_copy(
                w1_hbm.at[fidx_vm.at[i]], w1_vm.at[i], sem))
        for cp in copies:
            cp.wait()

        for g in range(_CB1 // 16):
            dacc = wsplat[_DENSE] + jnp.zeros((16,), jnp.float32)
            for k in range(_DENSE):
                dacc = dacc + dx_vm[k, pl.ds(g * 16, 16)] * wsplat[k]
            w1acc = dacc
            for f in range(_F):
                p = f * _CB1 + g * 16
                w1acc = w1acc + w1_vm[p // 128, pl.ds(p % 128, 16)]
            out_vm[pl.ds(g * 16, 16)] = w1acc
        pltpu.sync_copy(out_vm, out_hbm.at[pl.ds(base, _CB1)])
        return carry

    lax.fori_loop(0, _NCHUNK1, chunk, 0)


_fo_call = pl.kernel(
    _fo_body,
    out_type=jax.ShapeDtypeStruct((_B,), jnp.float32),
    mesh=plsc.VectorSubcoreMesh(core_axis_name="c", subcore_axis_name="s"),
    compiler_params=pltpu.CompilerParams(
        needs_layout_passes=False, use_tc_tiling_on_sc=False),
    scratch_types=[
        pltpu.VMEM((_F, _CB1), jnp.int32),
        pltpu.VMEM((_DENSE, _CB1), jnp.float32),
        pltpu.VMEM((_F * _CB1 // 128, 128), jnp.int32),
        pltpu.VMEM((_F * _CB1 // 128, 128), jnp.float32),
        pltpu.VMEM((16,), jnp.float32),
        pltpu.VMEM((_CB1,), jnp.float32),
        pltpu.SemaphoreType.DMA,
    ],
)


@jax.jit
def kernel(dense_x, discrete_x, dense_W, dense_b, w1_tables, emb_tables):
    idx_t = discrete_x.astype(jnp.int32).T
    idx_1d = idx_t.reshape(_F * _B)
    idx_3d = idx_t.reshape(_F, _B // 128, 128)
    dx_t = dense_x.T.reshape(_DENSE * _B)
    wb = jnp.concatenate([dense_W[:, 0], dense_b,
                          jnp.zeros((2,), jnp.float32)])
    emb_t = jnp.transpose(emb_tables, (0, 2, 1))
    emb_tail = jnp.pad(
        jnp.transpose(emb_tables[:, _VFULL * 128:, :], (0, 2, 1)),
        ((0, 0), (0, 0), (0, 128 - _VTAIL)))
    w1_flat = w1_tables.reshape(_F * _V)
    emb_lin = _tr_call(emb_t, emb_tail)
    out_so = _so_call(emb_lin.reshape(_F * _VROWS, 128), idx_3d)
    out_fo = _fo_call(w1_flat, idx_1d, dx_t, wb)
    res = out_fo + out_so[:, :4, :].reshape(_B)
    return res[:, None]


# speedup vs baseline: 1.1450x; 1.1450x over previous
import jax
import jax.numpy as jnp
from jax import lax
from jax.experimental import pallas as pl
from jax.experimental.pallas import tpu as pltpu
from jax.experimental.pallas import tpu_sc as plsc

_B = 16384
_F = 26
_V = 100000
_D = 16
_DENSE = 13

_NC = 2
_NS = 16
_NW = _NC * _NS
_RPW = _B // _NW

_VFULL = _V // 128
_VTAIL = _V - _VFULL * 128
_VROWS = 12504
_NBLK = _F * _VFULL


_NBUF = 4
_NIT = (_NBLK // _NW + _NBUF) // _NBUF


def _tr_body(embt_hbm, tail_hbm, out_hbm,
             vin0, vin1, vin2, vin3, vout0, vout1, vout2, vout3,
             rsem, wsem):
    wid = lax.axis_index("s") * _NC + lax.axis_index("c")
    lane = lax.iota(jnp.int32, 16)
    start = wid * _NBLK // _NW
    stop = (wid + 1) * _NBLK // _NW
    vin = [vin0, vin1, vin2, vin3]
    vout = [vout0, vout1, vout2, vout3]

    def issue_read(t, b):
        @pl.when(t < stop)
        def _():
            f = t // _VFULL
            c = lax.rem(t, _VFULL)
            pltpu.async_copy(
                embt_hbm.at[f, :, pl.ds(c * 128, 128)], vin[b], rsem)

    for b in range(_NBUF):
        issue_read(start + b, b)

    def ring(g, carry):
        t0 = start + g * _NBUF
        for b in range(_NBUF):
            t = t0 + b

            @pl.when(t < stop)
            def _(b=b, t=t):
                pltpu.make_async_copy(
                    embt_hbm.at[0, :, pl.ds(0, 128)], vin[b], rsem).wait()

                @pl.when(t - _NBUF >= start)
                def _():
                    pltpu.make_async_copy(
                        vout[b], out_hbm.at[0, pl.ds(0, 16), :], wsem).wait()

                for v in range(128):
                    cv = (lane + v) & 127
                    rv = lax.shift_right_logical(cv, 3)
                    ov = ((cv & 7) << 4) + lane
                    e = plsc.load_gather(vin[b], [lane, cv])
                    plsc.store_scatter(vout[b], [rv, ov], e)
                f = t // _VFULL
                c = lax.rem(t, _VFULL)
                pltpu.async_copy(
                    vout[b], out_hbm.at[f, pl.ds(c * 16, 16), :], wsem)
                issue_read(t + _NBUF, b)
        return carry

    lax.fori_loop(0, _NIT, ring, 0)
    for b in range(_NBUF):
        pltpu.make_async_copy(
            vout[b], out_hbm.at[0, pl.ds(0, 16), :], wsem).wait()

    @pl.when(wid < _F)
    def _():
        pltpu.async_copy(tail_hbm.at[wid], vin0, rsem).wait()
        for v in range(128):
            cv = (lane + v) & 127
            rv = lax.shift_right_logical(cv, 3)
            ov = ((cv & 7) << 4) + lane
            e = plsc.load_gather(vin0, [lane, cv])
            plsc.store_scatter(vout0, [rv, ov], e)
        pltpu.async_copy(vout0.at[pl.ds(0, 8)],
                         out_hbm.at[wid, pl.ds(_VFULL * 16, 8), :],
                         wsem).wait()


_tr_call = pl.kernel(
    _tr_body,
    out_type=jax.ShapeDtypeStruct((_F, _VROWS, 128), jnp.float32),
    mesh=plsc.VectorSubcoreMesh(core_axis_name="c", subcore_axis_name="s"),
    compiler_params=pltpu.CompilerParams(
        needs_layout_passes=False, use_tc_tiling_on_sc=True),
    scratch_types=(
        [pltpu.VMEM((16, 128), jnp.float32) for _ in range(2 * _NBUF)]
        + [pltpu.SemaphoreType.DMA, pltpu.SemaphoreType.DMA]
    ),
)


_CB2 = 16
_NCHUNK2 = _RPW // _CB2
_G2 = _F * _CB2


def _so_body(emb_hbm, idx_hbm, out_hbm,
             idx_vm, ridx_vm, off_vm, emb_vm, out_vm, sem):
    wid = lax.axis_index("s") * _NC + lax.axis_index("c")
    lane = lax.iota(jnp.int32, 16)
    half = wid // 2
    sub = (wid % 2) * 4

    cps = []
    for f in range(_F):
        cps.append(pltpu.async_copy(
            idx_hbm.at[f, pl.ds(half * 8, 8), :], idx_vm.at[f], sem))
    for cp in cps:
        cp.wait()

    def chunk(c, carry):
        lr = sub + c // 8
        col = (c % 8) * 16
        for f in range(_F):
            v = idx_vm[f, lr, pl.ds(col, 16)]
            off_vm[f, :] = (v & 7) * 16
            ridx_vm[f, :] = lax.shift_right_logical(v, 3) + f * _VROWS
        copies = []
        for f in range(_F):
            copies.append(pltpu.async_copy(
                emb_hbm.at[ridx_vm.at[f]],
                emb_vm.at[pl.ds(f * _CB2, _CB2)], sem))
        for cp in copies:
            cp.wait()

        eres = jnp.zeros((16,), jnp.float32)
        for j in range(_CB2):
            acc_s = jnp.zeros((16,), jnp.float32)
            acc_q = jnp.zeros((16,), jnp.float32)
            jsplat = jnp.full((16,), j, jnp.int32)
            for f in range(_F):
                osp = plsc.load_gather(
                    off_vm, [jnp.full((16,), f, jnp.int32), jsplat])
                e = plsc.load_gather(
                    emb_vm, [jnp.full((16,), f * _CB2 + j, jnp.int32),
                             osp + lane])
                acc_s = acc_s + e
                acc_q = acc_q + e * e
            r = 0.5 * jnp.sum(acc_s * acc_s - acc_q)
            eres = jnp.where(lane == j, eres + r, eres)
        out_vm[c // 8, pl.ds((c % 8) * 16, 16)] = eres
        return carry

    lax.fori_loop(0, _NCHUNK2, chunk, 0)
    pltpu.sync_copy(out_vm, out_hbm.at[wid])


_so_call = pl.kernel(
    _so_body,
    out_type=jax.ShapeDtypeStruct((_NW, 8, 128), jnp.float32),
    mesh=plsc.VectorSubcoreMesh(core_axis_name="c", subcore_axis_name="s"),
    compiler_params=pltpu.CompilerParams(
        needs_layout_passes=False, use_tc_tiling_on_sc=True),
    scratch_types=[
        pltpu.VMEM((_F, 8, 128), jnp.int32),
        pltpu.VMEM((_F, _CB2), jnp.int32),
        pltpu.VMEM((_F, _CB2), jnp.int32),
        pltpu.VMEM((_G2, 128), jnp.float32),
        pltpu.VMEM((8, 128), jnp.float32),
        pltpu.SemaphoreType.DMA,
    ],
)


_CB1 = 64
_NCHUNK1 = _RPW // _CB1


def _fo_body(w1_hbm, idx_hbm, dx_hbm, wb_hbm, out_hbm,
             idx_vm, dx_vm, fidx_vm, w1_vm, wb_vm, out_vm, sem):
    wid = lax.axis_index("s") * _NC + lax.axis_index("c")
    lane = lax.iota(jnp.int32, 16)

    pltpu.sync_copy(wb_hbm, wb_vm)
    wv = wb_vm[...]
    wsplat = [jnp.sum(jnp.where(lane == k, wv, 0.0))
              for k in range(_DENSE + 1)]

    def chunk(c, carry):
        base = wid * _RPW + c * _CB1
        cps_in = []
        for f in range(_F):
            cps_in.append(pltpu.async_copy(
                idx_hbm.at[pl.ds(f * _B + base, _CB1)], idx_vm.at[f], sem))
        for k in range(_DENSE):
            cps_in.append(pltpu.async_copy(
                dx_hbm.at[pl.ds(k * _B + base, _CB1)], dx_vm.at[k], sem))
        for cp in cps_in:
            cp.wait()

        for f in range(_F):
            for g in range(_CB1 // 16):
                v = idx_vm[f, pl.ds(g * 16, 16)] + f * _V
                p = f * _CB1 + g * 16
                fidx_vm[p // 128, pl.ds(p % 128, 16)] = v

        copies = []
        for i in range(_F * _CB1 // 128):
            copies.append(pltpu.async_copy(
                w1_hbm.at[fidx_vm.at[i]], w1_vm.at[i], sem))
        for cp in copies:
            cp.wait()

        for g in range(_CB1 // 16):
            dacc = wsplat[_DENSE] + jnp.zeros((16,), jnp.float32)
            for k in range(_DENSE):
                dacc = dacc + dx_vm[k, pl.ds(g * 16, 16)] * wsplat[k]
            w1acc = dacc
            for f in range(_F):
                p = f * _CB1 + g * 16
                w1acc = w1acc + w1_vm[p // 128, pl.ds(p % 128, 16)]
            out_vm[pl.ds(g * 16, 16)] = w1acc
        pltpu.sync_copy(out_vm, out_hbm.at[pl.ds(base, _CB1)])
        return carry

    lax.fori_loop(0, _NCHUNK1, chunk, 0)


_fo_call = pl.kernel(
    _fo_body,
    out_type=jax.ShapeDtypeStruct((_B,), jnp.float32),
    mesh=plsc.VectorSubcoreMesh(core_axis_name="c", subcore_axis_name="s"),
    compiler_params=pltpu.CompilerParams(
        needs_layout_passes=False, use_tc_tiling_on_sc=False),
    scratch_types=[
        pltpu.VMEM((_F, _CB1), jnp.int32),
        pltpu.VMEM((_DENSE, _CB1), jnp.float32),
        pltpu.VMEM((_F * _CB1 // 128, 128), jnp.int32),
        pltpu.VMEM((_F * _CB1 // 128, 128), jnp.float32),
        pltpu.VMEM((16,), jnp.float32),
        pltpu.VMEM((_CB1,), jnp.float32),
        pltpu.SemaphoreType.DMA,
    ],
)


@jax.jit
def kernel(dense_x, discrete_x, dense_W, dense_b, w1_tables, emb_tables):
    idx_t = discrete_x.astype(jnp.int32).T
    idx_1d = idx_t.reshape(_F * _B)
    idx_3d = idx_t.reshape(_F, _B // 128, 128)
    dx_t = dense_x.T.reshape(_DENSE * _B)
    wb = jnp.concatenate([dense_W[:, 0], dense_b,
                          jnp.zeros((2,), jnp.float32)])
    emb_t = jnp.transpose(emb_tables, (0, 2, 1))
    emb_tail = jnp.pad(
        jnp.transpose(emb_tables[:, _VFULL * 128:, :], (0, 2, 1)),
        ((0, 0), (0, 0), (0, 128 - _VTAIL)))
    w1_flat = w1_tables.reshape(_F * _V)
    emb_lin = _tr_call(emb_t, emb_tail)
    out_so = _so_call(emb_lin.reshape(_F * _VROWS, 128), idx_3d)
    out_fo = _fo_call(w1_flat, idx_1d, dx_t, wb)
    res = out_fo + out_so[:, :4, :].reshape(_B)
    return res[:, None]
